# baseline (speedup 1.0000x reference)
import jax
import jax.numpy as jnp
from jax import lax
from jax.experimental import pallas as pl
from jax.experimental.pallas import tpu as pltpu
from jax.experimental.pallas import tpu_sc as plsc

B, T, N, F_IN, F_OUT, DEG = 32, 12, 325, 64, 64, 16
BT = B * T
FA = 80
NC, NS = 2, 16
NW = NC * NS
BT_PER = BT // NW
ROWS = BT * N
RB = 2600


def _tc_body(h_ref, w_ref, tab_ref):
    tab_ref[...] = jnp.dot(h_ref[...], w_ref[...],
                           preferred_element_type=jnp.float32)


def _sc_body(tab_hbm, adj_hbm, out_hbm, adj_v, tab_v, out_v):
    cid = lax.axis_index("c")
    sid = lax.axis_index("s")
    wid = sid * NC + cid
    pltpu.sync_copy(adj_hbm, adj_v)
    col_s2 = jnp.full((16,), F_OUT, jnp.int32)

    def bt_body(k, carry):
        bt = wid * BT_PER + k
        pltpu.sync_copy(tab_hbm.at[bt], tab_v)

        def node_body(i, carry2):
            nbr = adj_v[i, :]
            svals = plsc.load_gather(tab_v, [nbr, col_s2])
            m = jnp.max(svals)
            ex = jnp.exp(svals - m)
            alpha = ex / jnp.sum(ex)
            accs = [jnp.zeros((16,), jnp.float32) for _ in range(4)]
            for j in range(DEG):
                aj = alpha[j]
                ij = nbr[j]
                for cb in range(4):
                    accs[cb] = accs[cb] + aj * tab_v[ij, pl.ds(cb * 16, 16)]
            for cb in range(4):
                out_v[i, pl.ds(cb * 16, 16)] = accs[cb]
            return carry2

        lax.fori_loop(0, N, node_body, 0)
        pltpu.sync_copy(out_v, out_hbm.at[bt])
        return carry

    lax.fori_loop(0, BT_PER, bt_body, 0)


def kernel(h, adj, W, a):
    h2 = h.reshape(ROWS, F_IN)
    wT = W.T
    a2 = a[F_OUT:]
    waug = jnp.concatenate(
        [wT, (wT @ a2)[:, None], jnp.zeros((F_IN, FA - F_OUT - 1), jnp.float32)],
        axis=1)

    tab = pl.pallas_call(
        _tc_body,
        grid=(ROWS // RB,),
        in_specs=[
            pl.BlockSpec((RB, F_IN), lambda i: (i, 0)),
            pl.BlockSpec((F_IN, FA), lambda i: (0, 0)),
        ],
        out_specs=pl.BlockSpec((RB, FA), lambda i: (i, 0)),
        out_shape=jax.ShapeDtypeStruct((ROWS, FA), jnp.float32),
    )(h2, waug)

    sc_fn = pl.kernel(
        _sc_body,
        out_type=jax.ShapeDtypeStruct((BT, N, F_OUT), jnp.float32),
        mesh=plsc.VectorSubcoreMesh(core_axis_name="c", subcore_axis_name="s",
                                    num_cores=NC, num_subcores=NS),
        compiler_params=pltpu.CompilerParams(needs_layout_passes=False),
        scratch_types=[
            pltpu.VMEM((N, DEG), jnp.int32),
            pltpu.VMEM((N, FA), jnp.float32),
            pltpu.VMEM((N, F_OUT), jnp.float32),
        ],
    )
    outp = sc_fn(tab.reshape(BT, N, FA), adj)
    return outp.reshape(B, T, N, F_OUT)
